# Initial kernel scaffold; baseline (speedup 1.0000x reference)
import jax
import jax.numpy as jnp
from jax.experimental import pallas as pl


def kernel(h, X, t_ij, edge_center, edge_neighbor, phi_ij, spharms, num_nodes, W_rs, mlp_W1, mlp_b1, mlp_W2, mlp_b2, ln_gamma, ln_beta, W_query, W_key):
    raise NotImplementedError("write your pallas kernel here")


# speedup vs baseline: 18.8317x; 18.8317x over previous
import functools

import jax
import jax.numpy as jnp
import numpy as np
from jax import lax
from jax.experimental import pallas as pl
from jax.experimental.pallas import tpu as pltpu
from jax.experimental.pallas import tpu_sc as plsc

N = 10000
E = 160000
LAT = 128
MULT = 8
HEAD = 16
SPH = 9
IRREP_DIMS = (1, 3, 5)
IRR = np.array([0, 1, 1, 1, 2, 2, 2, 2, 2])

ENVW = 3 * MULT
LINW = 3 * MULT * MULT
WEX = LAT + ENVW + LINW
GW = 640
CW = 128

NC = 2
NS = 16
NW = NC * NS
EPW = E // NW
ECH = 200
NCH = EPW // ECH


_PERM = np.concatenate([
    np.arange(LAT),
    np.array([i * MULT + m for m in range(MULT) for i in range(3)]),
    np.array([ENVW + i * 64 + n * MULT + m
              for m in range(MULT) for n in range(MULT) for i in range(3)]),
]).astype(np.int32)

_MASK4 = np.zeros((LAT, MULT), np.float32)
for _j in range(LAT):
    _MASK4[_j, _j // HEAD] = 4.0

_E3 = np.zeros((ENVW, MULT * SPH), np.float32)
for _m in range(MULT):
    for _s in range(SPH):
        _E3[_m * 3 + IRR[_s], _m * SPH + _s] = 1.0

_E24 = np.zeros((ENVW, MULT * SPH), np.float32)
for _n in range(MULT):
    for _s in range(SPH):
        _E24[_n * 3 + IRR[_s], _n * SPH + _s] = 1.0 / np.sqrt(MULT)

_T9 = np.zeros((SPH, MULT * SPH), np.float32)
for _k in range(MULT):
    for _s in range(SPH):
        _T9[_s, _k * SPH + _s] = 1.0

_R8 = np.zeros((MULT, MULT * SPH), np.float32)
for _m in range(MULT):
    for _s in range(SPH):
        _R8[_m, _m * SPH + _s] = 1.0

_IRRSUM = np.zeros((MULT * SPH, 3), np.float32)
for _m in range(MULT):
    for _s in range(SPH):
        _IRRSUM[_m * SPH + _s, IRR[_s]] = 1.0 / (MULT * IRREP_DIMS[IRR[_s]])

_RMSEXP = np.zeros((3, MULT * SPH), np.float32)
for _m in range(MULT):
    for _s in range(SPH):
        _RMSEXP[IRR[_s], _m * SPH + _s] = 1.0


def _silu(x):
    return x * jax.nn.sigmoid(x)


_BN = 400


def _p1_body(h_ref, xf_ref, w1_ref, b1_ref, w2_ref, b2_ref, wk_ref, out_ref):
    h = h_ref[...]
    a1 = _silu(jnp.dot(h, w1_ref[...], preferred_element_type=jnp.float32)
               + b1_ref[...])
    env = jnp.dot(a1, w2_ref[...], preferred_element_type=jnp.float32) + b2_ref[...]
    kn = jnp.dot(h, wk_ref[...], preferred_element_type=jnp.float32)
    out_ref[:, 0:WEX] = env
    out_ref[:, WEX:WEX + LAT] = kn
    out_ref[:, WEX + LAT:WEX + LAT + MULT * SPH] = xf_ref[...]
    out_ref[:, WEX + LAT + MULT * SPH:GW] = jnp.zeros(
        (_BN, GW - WEX - LAT - MULT * SPH), jnp.float32)


def _p1_call(h, xf, w1, b1, w2ex, b2ex, wk):
    full = lambda shape: pl.BlockSpec(shape, lambda i: (0, 0))
    return pl.pallas_call(
        _p1_body,
        grid=(N // _BN,),
        in_specs=[
            pl.BlockSpec((_BN, LAT), lambda i: (i, 0)),
            pl.BlockSpec((_BN, MULT * SPH), lambda i: (i, 0)),
            full((LAT, LAT)),
            full((1, LAT)),
            full((LAT, WEX)),
            full((1, WEX)),
            full((LAT, LAT)),
        ],
        out_specs=pl.BlockSpec((_BN, GW), lambda i: (i, 0)),
        out_shape=jax.ShapeDtypeStruct((N, GW), jnp.float32),
    )(h, xf, w1, b1, w2ex, b2ex, wk)


def _sc_mesh():
    return plsc.VectorSubcoreMesh(core_axis_name="c", subcore_axis_name="s",
                                  num_cores=NC, num_subcores=NS)


@functools.cache
def _build_p2():
    @functools.partial(
        pl.kernel,
        out_type=jax.ShapeDtypeStruct((E, GW), jnp.float32),
        mesh=_sc_mesh(),
        scratch_types=[
            pltpu.VMEM((ECH,), jnp.int32),
            pltpu.VMEM((ECH, GW), jnp.float32),
            pltpu.SemaphoreType.DMA,
        ],
    )
    def p2(table_hbm, idx_hbm, out_hbm, idx_v, rows_v, sem):
        wid = lax.axis_index("s") * NC + lax.axis_index("c")

        def step(k, carry):
            base = wid * EPW + k * ECH
            pltpu.sync_copy(idx_hbm.at[pl.ds(base, ECH)], idx_v)
            pltpu.async_copy(table_hbm.at[idx_v], rows_v, sem).wait()
            pltpu.sync_copy(rows_v, out_hbm.at[pl.ds(base, ECH)])
            return carry

        lax.fori_loop(0, NCH, step, 0)

    return p2


def _p2_gather(table, idx):
    return _build_p2()(table, idx)


_BE = 640


def _p3_body(t_ref, g_ref, sph_ref, wcat_ref, mask4_ref, e3_ref, e24_ref,
             t9_ref, r8_ref, c1_ref, c2_ref):
    t = t_ref[...]
    g = g_ref[...]
    tq = jnp.dot(t, wcat_ref[...], preferred_element_type=jnp.float32)
    s = tq[:, 0:WEX]
    q = tq[:, WEX:WEX + LAT]
    envj = g[:, 0:WEX]
    kn = g[:, WEX:WEX + LAT]
    xg = g[:, WEX + LAT:WEX + LAT + MULT * SPH]
    env_w = s * envj
    wlog = jnp.dot(q * kn, mask4_ref[...], preferred_element_type=jnp.float32)
    ex = jnp.exp(wlog)
    sph_rep = jnp.dot(sph_ref[...], t9_ref[...], preferred_element_type=jnp.float32)
    dsp = jnp.dot(env_w[:, LAT:LAT + ENVW], e3_ref[...],
                  preferred_element_type=jnp.float32) * sph_rep
    e24 = e24_ref[...]
    t9 = t9_ref[...]
    eq = dsp
    for m in range(MULT):
        wm = env_w[:, LAT + ENVW + ENVW * m:LAT + ENVW + ENVW * (m + 1)]
        xm = xg[:, SPH * m:SPH * (m + 1)]
        eq = eq + (jnp.dot(wm, e24, preferred_element_type=jnp.float32)
                   * jnp.dot(xm, t9, preferred_element_type=jnp.float32))
    numer = eq * jnp.dot(ex, r8_ref[...], preferred_element_type=jnp.float32)
    c1_ref[...] = env_w[:, 0:LAT]
    c2_ref[:, 0:MULT * SPH] = numer
    c2_ref[:, MULT * SPH:MULT * SPH + MULT] = ex
    c2_ref[:, MULT * SPH + MULT:CW] = jnp.zeros(
        (_BE, CW - MULT * SPH - MULT), jnp.float32)


def _p3_call(t_ij, g, sph, wcat, consts):
    full = lambda shape: pl.BlockSpec(shape, lambda i: (0, 0))
    return pl.pallas_call(
        _p3_body,
        grid=(E // _BE,),
        in_specs=[
            pl.BlockSpec((_BE, LAT), lambda i: (i, 0)),
            pl.BlockSpec((_BE, GW), lambda i: (i, 0)),
            pl.BlockSpec((_BE, SPH), lambda i: (i, 0)),
            full((LAT, WEX + LAT)),
            full((LAT, MULT)),
            full((ENVW, MULT * SPH)),
            full((ENVW, MULT * SPH)),
            full((SPH, MULT * SPH)),
            full((MULT, MULT * SPH)),
        ],
        out_specs=(
            pl.BlockSpec((_BE, CW), lambda i: (i, 0)),
            pl.BlockSpec((_BE, CW), lambda i: (i, 0)),
        ),
        out_shape=(
            jax.ShapeDtypeStruct((E, CW), jnp.float32),
            jax.ShapeDtypeStruct((E, CW), jnp.float32),
        ),
    )(t_ij, g, sph, wcat, *consts)


@functools.cache
def _build_p4():
    @functools.partial(
        pl.kernel,
        out_type=jax.ShapeDtypeStruct((NC, N, CW), jnp.float32),
        mesh=_sc_mesh(),
        scratch_types=[
            pltpu.VMEM((ECH,), jnp.int32),
            pltpu.VMEM((ECH, CW), jnp.float32),
            pltpu.VMEM_SHARED((N, CW), jnp.float32),
            pltpu.SemaphoreType.DMA,
        ],
    )
    def p4(c_hbm, ctr_hbm, zeros_hbm, out_hbm, idx_v, c_v, acc, sem):
        cid = lax.axis_index("c")
        sid = lax.axis_index("s")
        wid = sid * NC + cid

        @pl.when(sid == 0)
        def _init():
            pltpu.sync_copy(zeros_hbm, acc)

        plsc.subcore_barrier()

        def step(k, carry):
            base = wid * EPW + k * ECH
            pltpu.sync_copy(ctr_hbm.at[pl.ds(base, ECH)], idx_v)
            pltpu.sync_copy(c_hbm.at[pl.ds(base, ECH)], c_v)
            pltpu.sync_copy(c_v, acc.at[idx_v], add=True)
            return carry

        lax.fori_loop(0, NCH, step, 0)
        plsc.subcore_barrier()

        @pl.when(sid == 0)
        def _flush():
            pltpu.sync_copy(acc, out_hbm.at[cid])

    return p4


def _p4_scatter(c, ctr, zeros):
    return _build_p4()(c, ctr, zeros)


def _p5_body(h_ref, xf_ref, p0a_ref, p1a_ref, p0b_ref, p1b_ref, g_ref, b_ref,
             r8_ref, irs_ref, rme_ref, ho_ref, xo_ref):
    hn = h_ref[...] + p0a_ref[...] + p1a_ref[...]
    mu = jnp.mean(hn, axis=1, keepdims=True)
    xc = hn - mu
    var = jnp.mean(xc * xc, axis=1, keepdims=True)
    ho_ref[...] = xc * lax.rsqrt(var + 1e-5) * g_ref[...] + b_ref[...]
    accb = p0b_ref[...] + p1b_ref[...]
    numer = accb[:, 0:MULT * SPH]
    den = accb[:, MULT * SPH:MULT * SPH + MULT]
    den_rep = jnp.dot(den, r8_ref[...], preferred_element_type=jnp.float32)
    xn = xf_ref[...] + numer / (den_rep + 1e-16)
    ms = jnp.dot(xn * xn, irs_ref[...], preferred_element_type=jnp.float32)
    inv = lax.rsqrt(ms + 1e-8)
    xo_ref[...] = xn * jnp.dot(inv, rme_ref[...], preferred_element_type=jnp.float32)


def _p5_call(h, xf, p0a, p1a, p0b, p1b, gamma, beta, consts):
    full = lambda shape: pl.BlockSpec(shape, lambda i: (0, 0))
    return pl.pallas_call(
        _p5_body,
        grid=(N // _BN,),
        in_specs=[
            pl.BlockSpec((_BN, LAT), lambda i: (i, 0)),
            pl.BlockSpec((_BN, MULT * SPH), lambda i: (i, 0)),
            pl.BlockSpec((_BN, CW), lambda i: (i, 0)),
            pl.BlockSpec((_BN, CW), lambda i: (i, 0)),
            pl.BlockSpec((_BN, CW), lambda i: (i, 0)),
            pl.BlockSpec((_BN, CW), lambda i: (i, 0)),
            full((1, LAT)),
            full((1, LAT)),
            full((MULT, MULT * SPH)),
            full((MULT * SPH, 3)),
            full((3, MULT * SPH)),
        ],
        out_specs=(
            pl.BlockSpec((_BN, LAT), lambda i: (i, 0)),
            pl.BlockSpec((_BN, MULT * SPH), lambda i: (i, 0)),
        ),
        out_shape=(
            jax.ShapeDtypeStruct((N, LAT), jnp.float32),
            jax.ShapeDtypeStruct((N, MULT * SPH), jnp.float32),
        ),
    )(h, xf, p0a, p1a, p0b, p1b, gamma, beta, *consts)


def kernel(h, X, t_ij, edge_center, edge_neighbor, phi_ij, spharms, num_nodes,
           W_rs, mlp_W1, mlp_b1, mlp_W2, mlp_b2, ln_gamma, ln_beta,
           W_query, W_key):
    del phi_ij, num_nodes
    xf = X.reshape(N, MULT * SPH)
    perm = jnp.asarray(_PERM)
    w_rs_ex = W_rs[:, perm]
    w2ex = mlp_W2[:, perm]
    b2ex = mlp_b2[perm].reshape(1, WEX)
    wcat = jnp.concatenate([w_rs_ex, W_query], axis=1)

    table = _p1_call(h, xf, mlp_W1, mlp_b1.reshape(1, LAT), w2ex, b2ex, W_key)
    g = _p2_gather(table, jnp.asarray(edge_neighbor, jnp.int32))
    consts3 = [jnp.asarray(c) for c in (_MASK4, _E3, _E24, _T9, _R8)]
    c1, c2 = _p3_call(t_ij, g, spharms, wcat, consts3)
    ctr = jnp.asarray(edge_center, jnp.int32)
    zeros = jnp.zeros((N, CW), jnp.float32)
    parta = _p4_scatter(c1, ctr, zeros)
    partb = _p4_scatter(c2, ctr, zeros)
    consts5 = [jnp.asarray(c_) for c_ in (_R8, _IRRSUM, _RMSEXP)]
    h_out, xf_out = _p5_call(h, xf, parta[0], parta[1], partb[0], partb[1],
                             ln_gamma.reshape(1, LAT), ln_beta.reshape(1, LAT),
                             consts5)
    return (h_out, xf_out.reshape(N, MULT, SPH), t_ij)
